# baseline (speedup 1.0000x reference)
import jax
import jax.numpy as jnp
from jax import lax
from jax.experimental import pallas as pl
from jax.experimental.pallas import tpu as pltpu
from jax.experimental.pallas import tpu_sc as plsc

HIDDEN = 4096
EXPERTS = 64
K = 8
TOKENS = 16384
BLOCK_TOKENS = 1024

NW = 32
TPW = TOKENS // NW
GRP = 16
NGRP = TPW // GRP


def _matmul_block(x_ref, w_ref, logits_ref):
    logits_ref[...] = jnp.dot(x_ref[...], w_ref[...],
                              preferred_element_type=jnp.float32)


@jax.jit
def _logits_tc(x2d, W):
    return pl.pallas_call(
        _matmul_block,
        grid=(TOKENS // BLOCK_TOKENS,),
        in_specs=[
            pl.BlockSpec((BLOCK_TOKENS, HIDDEN), lambda i: (i, 0)),
            pl.BlockSpec((HIDDEN, EXPERTS), lambda i: (0, 0)),
        ],
        out_specs=pl.BlockSpec((BLOCK_TOKENS, EXPERTS), lambda i: (i, 0)),
        out_shape=jax.ShapeDtypeStruct((TOKENS, EXPERTS), jnp.float32),
    )(x2d, W)


def _sc_topk_body(lflat_hbm, wflat_hbm, iflat_hbm, lbuf, wbuf, ibuf):
    wid = lax.axis_index("s") * 2 + lax.axis_index("c")
    g0 = wid * NGRP

    def group(g, carry):
        lbase = (g0 + g) * (EXPERTS * GRP)
        obase = (g0 + g) * (K * GRP)
        pltpu.sync_copy(lflat_hbm.at[pl.ds(lbase, EXPERTS * GRP)], lbuf)

        top_v = [jnp.full((GRP,), -jnp.inf, jnp.float32) for _ in range(K)]
        top_i = [jnp.zeros((GRP,), jnp.int32) for _ in range(K)]
        for e in range(EXPERTS):
            v = lbuf[pl.ds(e * GRP, GRP)]
            vi = jnp.full((GRP,), e, jnp.int32)
            for j in range(K):
                m = v > top_v[j]
                tv = jnp.where(m, v, top_v[j])
                v = jnp.where(m, top_v[j], v)
                ti = jnp.where(m, vi, top_i[j])
                vi = jnp.where(m, top_i[j], vi)
                top_v[j] = tv
                top_i[j] = ti

        ew = [jnp.exp(tv - top_v[0]) for tv in top_v]
        s = ew[0]
        for j in range(1, K):
            s = s + ew[j]
        rs = 1.0 / s
        for j in range(K):
            wbuf[pl.ds(j * GRP, GRP)] = ew[j] * rs
            ibuf[pl.ds(j * GRP, GRP)] = top_i[j]

        pltpu.sync_copy(wbuf, wflat_hbm.at[pl.ds(obase, K * GRP)])
        pltpu.sync_copy(ibuf, iflat_hbm.at[pl.ds(obase, K * GRP)])
        return carry

    lax.fori_loop(0, NGRP, group, 0)


_sc_topk = pl.kernel(
    _sc_topk_body,
    mesh=plsc.VectorSubcoreMesh(core_axis_name="c", subcore_axis_name="s"),
    out_type=[
        jax.ShapeDtypeStruct((TOKENS * K,), jnp.float32),
        jax.ShapeDtypeStruct((TOKENS * K,), jnp.int32),
    ],
    scratch_types=[
        pltpu.VMEM((EXPERTS * GRP,), jnp.float32),
        pltpu.VMEM((K * GRP,), jnp.float32),
        pltpu.VMEM((K * GRP,), jnp.int32),
    ],
)


def kernel(hidden_states, W):
    batch, seq, hidden = hidden_states.shape
    x2d = hidden_states.reshape(batch * seq, hidden)
    logits = _logits_tc(x2d, W)
    lflat = logits.reshape(TOKENS // GRP, GRP, EXPERTS)
    lflat = lflat.transpose(0, 2, 1).reshape(-1)
    wflat, iflat = _sc_topk(lflat)
    weights = wflat.reshape(TOKENS // GRP, K, GRP).transpose(0, 2, 1)
    idx = iflat.reshape(TOKENS // GRP, K, GRP).transpose(0, 2, 1)
    return (
        weights.reshape(batch, seq, K),
        idx.reshape(batch, seq, K),
        logits.reshape(batch, seq, EXPERTS),
    )


# speedup vs baseline: 1.6939x; 1.6939x over previous
import functools

import jax
import jax.numpy as jnp
from jax import lax
from jax.experimental import pallas as pl

HIDDEN = 4096
EXPERTS = 64
K = 8
BLOCK_TOKENS = 1024
CHUNKS = 4


def _topk8_t(logits_t):
    b = logits_t.shape[1]
    iota = lax.broadcasted_iota(jnp.int32, (EXPERTS, b), 0).astype(jnp.float32)
    neg_inf = jnp.float32(-jnp.inf)

    vals = logits_t
    top_v = []
    top_i = []
    for _ in range(K):
        m = jnp.max(vals, axis=0, keepdims=True)
        idx = jnp.min(jnp.where(vals == m, iota, jnp.float32(EXPERTS)),
                      axis=0, keepdims=True)
        top_v.append(m)
        top_i.append(idx)
        vals = jnp.where(iota == idx, neg_inf, vals)

    tv = jnp.concatenate(top_v, axis=0)
    ti = jnp.concatenate(top_i, axis=0)
    ew = jnp.exp(tv - tv[:1])
    w = ew / jnp.sum(ew, axis=0, keepdims=True)
    return w.T, ti.T.astype(jnp.int32)


def _router_block(x_ref, w_ref, logits_ref, weights_ref, idx_ref):
    w = w_ref[...]
    c = BLOCK_TOKENS // CHUNKS
    for i in range(CHUNKS):
        rows = pl.ds(i * c, c)
        x = x_ref[rows, :]
        logits_ref[rows, :] = jnp.dot(x, w, preferred_element_type=jnp.float32)
        logits_t = lax.dot_general(w, x, (((0,), (1,)), ((), ())),
                                   preferred_element_type=jnp.float32)
        wts, idx = _topk8_t(logits_t)
        weights_ref[rows, :] = wts
        idx_ref[rows, :] = idx


@functools.partial(jax.jit, static_argnames=())
def _router(x2d, W):
    n = x2d.shape[0]
    grid = (n // BLOCK_TOKENS,)
    return pl.pallas_call(
        _router_block,
        grid=grid,
        in_specs=[
            pl.BlockSpec((BLOCK_TOKENS, HIDDEN), lambda i: (i, 0)),
            pl.BlockSpec((HIDDEN, EXPERTS), lambda i: (0, 0)),
        ],
        out_specs=[
            pl.BlockSpec((BLOCK_TOKENS, EXPERTS), lambda i: (i, 0)),
            pl.BlockSpec((BLOCK_TOKENS, K), lambda i: (i, 0)),
            pl.BlockSpec((BLOCK_TOKENS, K), lambda i: (i, 0)),
        ],
        out_shape=[
            jax.ShapeDtypeStruct((n, EXPERTS), jnp.float32),
            jax.ShapeDtypeStruct((n, K), jnp.float32),
            jax.ShapeDtypeStruct((n, K), jnp.int32),
        ],
    )(x2d, W)


def kernel(hidden_states, W):
    batch, seq, hidden = hidden_states.shape
    x2d = hidden_states.reshape(batch * seq, hidden)
    logits, weights, idx = _router(x2d, W)
    return (
        weights.reshape(batch, seq, K),
        idx.reshape(batch, seq, K),
        logits.reshape(batch, seq, EXPERTS),
    )


# speedup vs baseline: 1.8225x; 1.0759x over previous
import functools

import jax
import jax.numpy as jnp
from jax import lax
from jax.experimental import pallas as pl

HIDDEN = 4096
EXPERTS = 64
K = 8
BLOCK_TOKENS = 1024
CHUNKS = 4


def _topk8_t(logits_t):
    b = logits_t.shape[1]
    iota = lax.broadcasted_iota(jnp.int32, (EXPERTS, b), 0).astype(jnp.float32)
    neg_inf = jnp.float32(-jnp.inf)

    vals = logits_t
    top_v = []
    top_i = []
    for _ in range(K):
        m = jnp.max(vals, axis=0, keepdims=True)
        idx = jnp.min(jnp.where(vals == m, iota, jnp.float32(EXPERTS)),
                      axis=0, keepdims=True)
        top_v.append(m)
        top_i.append(idx)
        vals = jnp.where(iota == idx, neg_inf, vals)

    tv = jnp.concatenate(top_v, axis=0)
    ti = jnp.concatenate(top_i, axis=0)
    ew = jnp.exp(tv - tv[:1])
    w = ew / jnp.sum(ew, axis=0, keepdims=True)
    return w.T, ti.T.astype(jnp.int32)


def _router_block(x_ref, w_ref, logits_ref, weights_ref, idx_ref):
    w = w_ref[...]
    c = BLOCK_TOKENS // CHUNKS
    for i in range(CHUNKS):
        rows = pl.ds(i * c, c)
        x = x_ref[rows, :]
        logits_t = lax.dot_general(w, x, (((0,), (1,)), ((), ())),
                                   preferred_element_type=jnp.float32)
        logits_ref[rows, :] = logits_t.T
        wts, idx = _topk8_t(logits_t)
        weights_ref[rows, :] = wts
        idx_ref[rows, :] = idx


@functools.partial(jax.jit, static_argnames=())
def _router(x2d, W):
    n = x2d.shape[0]
    grid = (n // BLOCK_TOKENS,)
    return pl.pallas_call(
        _router_block,
        grid=grid,
        in_specs=[
            pl.BlockSpec((BLOCK_TOKENS, HIDDEN), lambda i: (i, 0)),
            pl.BlockSpec((HIDDEN, EXPERTS), lambda i: (0, 0)),
        ],
        out_specs=[
            pl.BlockSpec((BLOCK_TOKENS, EXPERTS), lambda i: (i, 0)),
            pl.BlockSpec((BLOCK_TOKENS, K), lambda i: (i, 0)),
            pl.BlockSpec((BLOCK_TOKENS, K), lambda i: (i, 0)),
        ],
        out_shape=[
            jax.ShapeDtypeStruct((n, EXPERTS), jnp.float32),
            jax.ShapeDtypeStruct((n, K), jnp.float32),
            jax.ShapeDtypeStruct((n, K), jnp.int32),
        ],
    )(x2d, W)


def kernel(hidden_states, W):
    batch, seq, hidden = hidden_states.shape
    x2d = hidden_states.reshape(batch * seq, hidden)
    logits, weights, idx = _router(x2d, W)
    return (
        weights.reshape(batch, seq, K),
        idx.reshape(batch, seq, K),
        logits.reshape(batch, seq, EXPERTS),
    )


# speedup vs baseline: 1.8268x; 1.0024x over previous
import functools

import jax
import jax.numpy as jnp
from jax import lax
from jax.experimental import pallas as pl

HIDDEN = 4096
EXPERTS = 64
K = 8
BLOCK_TOKENS = 1024
CHUNKS = 2


def _topk8_t(logits_t):
    b = logits_t.shape[1]
    iota = lax.broadcasted_iota(jnp.int32, (EXPERTS, b), 0).astype(jnp.float32)
    neg_inf = jnp.float32(-jnp.inf)

    vals = logits_t
    top_v = []
    top_i = []
    for _ in range(K):
        m = jnp.max(vals, axis=0, keepdims=True)
        idx = jnp.min(jnp.where(vals == m, iota, jnp.float32(EXPERTS)),
                      axis=0, keepdims=True)
        top_v.append(m)
        top_i.append(idx)
        vals = jnp.where(iota == idx, neg_inf, vals)

    tv = jnp.concatenate(top_v, axis=0)
    ti = jnp.concatenate(top_i, axis=0)
    ew = jnp.exp(tv - tv[:1])
    w = ew / jnp.sum(ew, axis=0, keepdims=True)
    return w.T, ti.T.astype(jnp.int32)


def _router_block(x_ref, w_ref, logits_ref, weights_ref, idx_ref):
    w = w_ref[...]
    c = BLOCK_TOKENS // CHUNKS
    for i in range(CHUNKS):
        rows = pl.ds(i * c, c)
        x = x_ref[rows, :]
        logits_t = lax.dot_general(w, x, (((0,), (1,)), ((), ())),
                                   preferred_element_type=jnp.float32)
        logits_ref[rows, :] = logits_t.T
        wts, idx = _topk8_t(logits_t)
        weights_ref[rows, :] = wts
        idx_ref[rows, :] = idx


@functools.partial(jax.jit, static_argnames=())
def _router(x2d, W):
    n = x2d.shape[0]
    grid = (n // BLOCK_TOKENS,)
    return pl.pallas_call(
        _router_block,
        grid=grid,
        in_specs=[
            pl.BlockSpec((BLOCK_TOKENS, HIDDEN), lambda i: (i, 0)),
            pl.BlockSpec((HIDDEN, EXPERTS), lambda i: (0, 0)),
        ],
        out_specs=[
            pl.BlockSpec((BLOCK_TOKENS, EXPERTS), lambda i: (i, 0)),
            pl.BlockSpec((BLOCK_TOKENS, K), lambda i: (i, 0)),
            pl.BlockSpec((BLOCK_TOKENS, K), lambda i: (i, 0)),
        ],
        out_shape=[
            jax.ShapeDtypeStruct((n, EXPERTS), jnp.float32),
            jax.ShapeDtypeStruct((n, K), jnp.float32),
            jax.ShapeDtypeStruct((n, K), jnp.int32),
        ],
    )(x2d, W)


def kernel(hidden_states, W):
    batch, seq, hidden = hidden_states.shape
    x2d = hidden_states.reshape(batch * seq, hidden)
    logits, weights, idx = _router(x2d, W)
    return (
        weights.reshape(batch, seq, K),
        idx.reshape(batch, seq, K),
        logits.reshape(batch, seq, EXPERTS),
    )


# speedup vs baseline: 1.8331x; 1.0035x over previous
import functools

import jax
import jax.numpy as jnp
from jax import lax
from jax.experimental import pallas as pl

HIDDEN = 4096
EXPERTS = 64
K = 8
BLOCK_TOKENS = 1024
CHUNKS = 1


def _topk8_t(logits_t):
    b = logits_t.shape[1]
    iota = lax.broadcasted_iota(jnp.int32, (EXPERTS, b), 0).astype(jnp.float32)
    neg_inf = jnp.float32(-jnp.inf)

    vals = logits_t
    top_v = []
    top_i = []
    for _ in range(K):
        m = jnp.max(vals, axis=0, keepdims=True)
        idx = jnp.min(jnp.where(vals == m, iota, jnp.float32(EXPERTS)),
                      axis=0, keepdims=True)
        top_v.append(m)
        top_i.append(idx)
        vals = jnp.where(iota == idx, neg_inf, vals)

    tv = jnp.concatenate(top_v, axis=0)
    ti = jnp.concatenate(top_i, axis=0)
    ew = jnp.exp(tv - tv[:1])
    w = ew / jnp.sum(ew, axis=0, keepdims=True)
    return w.T, ti.T.astype(jnp.int32)


def _router_block(x_ref, w_ref, logits_ref, weights_ref, idx_ref):
    w = w_ref[...]
    c = BLOCK_TOKENS // CHUNKS
    for i in range(CHUNKS):
        rows = pl.ds(i * c, c)
        x = x_ref[rows, :]
        logits_t = lax.dot_general(w, x, (((0,), (1,)), ((), ())),
                                   preferred_element_type=jnp.float32)
        logits_ref[rows, :] = logits_t.T
        wts, idx = _topk8_t(logits_t)
        weights_ref[rows, :] = wts
        idx_ref[rows, :] = idx


@functools.partial(jax.jit, static_argnames=())
def _router(x2d, W):
    n = x2d.shape[0]
    grid = (n // BLOCK_TOKENS,)
    return pl.pallas_call(
        _router_block,
        grid=grid,
        in_specs=[
            pl.BlockSpec((BLOCK_TOKENS, HIDDEN), lambda i: (i, 0)),
            pl.BlockSpec((HIDDEN, EXPERTS), lambda i: (0, 0)),
        ],
        out_specs=[
            pl.BlockSpec((BLOCK_TOKENS, EXPERTS), lambda i: (i, 0)),
            pl.BlockSpec((BLOCK_TOKENS, K), lambda i: (i, 0)),
            pl.BlockSpec((BLOCK_TOKENS, K), lambda i: (i, 0)),
        ],
        out_shape=[
            jax.ShapeDtypeStruct((n, EXPERTS), jnp.float32),
            jax.ShapeDtypeStruct((n, K), jnp.float32),
            jax.ShapeDtypeStruct((n, K), jnp.int32),
        ],
    )(x2d, W)


def kernel(hidden_states, W):
    batch, seq, hidden = hidden_states.shape
    x2d = hidden_states.reshape(batch * seq, hidden)
    logits, weights, idx = _router(x2d, W)
    return (
        weights.reshape(batch, seq, K),
        idx.reshape(batch, seq, K),
        logits.reshape(batch, seq, EXPERTS),
    )
